# Initial kernel scaffold; baseline (speedup 1.0000x reference)
import jax
import jax.numpy as jnp
from jax.experimental import pallas as pl


def kernel(x, prototypes):
    raise NotImplementedError("write your pallas kernel here")


# speedup vs baseline: 10.8955x; 10.8955x over previous
import functools

import jax
import jax.numpy as jnp
from jax import lax
from jax.experimental import pallas as pl
from jax.experimental.pallas import tpu as pltpu
from jax.experimental.pallas import tpu_sc as plsc

_D = 256
_NP = 8192
_M = 64 * 576
_COMMIT = 0.25

_BM = 256
_G = _M // _BM


def _dist_argmin_body(x_ref, p_ref, idx_ref, pn_ref):
    i = pl.program_id(0)

    @pl.when(i == 0)
    def _():
        p = p_ref[...]
        pn_ref[...] = jnp.sum(p * p, axis=1)[None, :]

    x = x_ref[...]
    xn = jnp.sum(x * x, axis=1, keepdims=True)
    mm = lax.dot_general(x, p_ref[...], (((1,), (1,)), ((), ())),
                         preferred_element_type=jnp.float32)
    dist = xn + pn_ref[...] - 2.0 * mm
    m = jnp.min(dist, axis=1, keepdims=True)
    lane = lax.broadcasted_iota(jnp.int32, dist.shape, 1)
    idx = jnp.min(jnp.where(dist == m, lane, _NP), axis=1)
    idx_ref[...] = idx[None, None, :]


_dist_argmin = pl.pallas_call(
    _dist_argmin_body,
    grid=(_G,),
    in_specs=[
        pl.BlockSpec((_BM, _D), lambda i: (i, 0)),
        pl.BlockSpec((_NP, _D), lambda i: (0, 0)),
    ],
    out_specs=pl.BlockSpec((1, 1, _BM), lambda i: (i, 0, 0)),
    out_shape=jax.ShapeDtypeStruct((_G, 1, _BM), jnp.int32),
    scratch_shapes=[pltpu.VMEM((1, _NP), jnp.float32)],
)

_NC, _NS = 2, 16
_NW = _NC * _NS
_RPW = _M // _NW
_CH = 128
_NCH = _RPW // _CH

@functools.cache
def _build_gather_sc():
    mesh = plsc.VectorSubcoreMesh(core_axis_name="c", subcore_axis_name="s")

    @functools.partial(
        pl.kernel,
        out_type=jax.ShapeDtypeStruct((_M, _D), jnp.float32),
        mesh=mesh,
        scratch_types=[
            pltpu.VMEM((_CH,), jnp.int32),
            pltpu.VMEM((_CH, _D), jnp.float32),
            pltpu.SemaphoreType.DMA,
        ],
    )
    def gather_sc(idx_hbm, table_hbm, out_hbm, idx_v, rows_v, sem):
        wid = lax.axis_index("s") * _NC + lax.axis_index("c")
        base = wid * _RPW
        for j in range(_NCH):
            off = base + j * _CH
            pltpu.sync_copy(idx_hbm.at[pl.ds(off, _CH)], idx_v)
            pltpu.async_copy(table_hbm.at[idx_v], rows_v, sem).wait()
            pltpu.sync_copy(rows_v, out_hbm.at[pl.ds(off, _CH)])

    return gather_sc


_BR = 2048
_GR = _M // _BR


def _finish_body(x_ref, p_ref, pst_ref, res_ref, ss_ref):
    i = pl.program_id(0)
    x = x_ref[...]
    p = p_ref[...]
    t = p - x
    pst = x + t
    pst_ref[...] = pst
    res_ref[...] = x - pst

    @pl.when(i == 0)
    def _():
        ss_ref[...] = jnp.zeros_like(ss_ref)

    ss_ref[...] += jnp.sum(t * t).reshape(1, 1)

    @pl.when(i == _GR - 1)
    def _():
        ss_ref[...] = ss_ref[...] * ((1.0 + _COMMIT) / (_M * _D))


_finish = pl.pallas_call(
    _finish_body,
    grid=(_GR,),
    in_specs=[
        pl.BlockSpec((_BR, _D), lambda i: (i, 0)),
        pl.BlockSpec((_BR, _D), lambda i: (i, 0)),
    ],
    out_specs=[
        pl.BlockSpec((_BR, _D), lambda i: (i, 0)),
        pl.BlockSpec((_BR, _D), lambda i: (i, 0)),
        pl.BlockSpec((1, 1), lambda i: (0, 0)),
    ],
    out_shape=[
        jax.ShapeDtypeStruct((_M, _D), jnp.float32),
        jax.ShapeDtypeStruct((_M, _D), jnp.float32),
        jax.ShapeDtypeStruct((1, 1), jnp.float32),
    ],
)


def kernel(x, prototypes):
    x_shape = x.shape
    xf = x.reshape(_M, _D)
    idx = _dist_argmin(xf, prototypes).reshape(_M)
    proto = _build_gather_sc()(idx, prototypes)
    pst, res, loss = _finish(xf, proto)
    return (pst.reshape(x_shape), res.reshape(x_shape), loss.reshape(()))


# speedup vs baseline: 13.6704x; 1.2547x over previous
import functools

import jax
import jax.numpy as jnp
from jax import lax
from jax.experimental import pallas as pl
from jax.experimental.pallas import tpu as pltpu
from jax.experimental.pallas import tpu_sc as plsc

_D = 256
_NP = 8192
_M = 64 * 576
_COMMIT = 0.25

_BM = 256
_G = _M // _BM


def _dist_argmin_body(x_ref, p_ref, idx_ref, pn_ref):
    i = pl.program_id(0)

    @pl.when(i == 0)
    def _():
        p = p_ref[...]
        pn_ref[...] = jnp.sum(p * p, axis=1)[None, :]

    x = x_ref[...]
    xn = jnp.sum(x * x, axis=1, keepdims=True)
    mm = lax.dot_general(x * -2.0, p_ref[...], (((1,), (1,)), ((), ())),
                         preferred_element_type=jnp.float32)
    dist = (xn + pn_ref[...]) + mm
    idx = jnp.argmin(dist, axis=1).astype(jnp.int32)
    idx_ref[...] = idx[None, None, :]


_dist_argmin = pl.pallas_call(
    _dist_argmin_body,
    grid=(_G,),
    in_specs=[
        pl.BlockSpec((_BM, _D), lambda i: (i, 0)),
        pl.BlockSpec((_NP, _D), lambda i: (0, 0)),
    ],
    out_specs=pl.BlockSpec((1, 1, _BM), lambda i: (i, 0, 0)),
    out_shape=jax.ShapeDtypeStruct((_G, 1, _BM), jnp.int32),
    scratch_shapes=[pltpu.VMEM((1, _NP), jnp.float32)],
)

_NC, _NS = 2, 16
_NW = _NC * _NS
_RPW = _M // _NW
_CH = 128
_NCH = _RPW // _CH

@functools.cache
def _build_gather_sc():
    mesh = plsc.VectorSubcoreMesh(core_axis_name="c", subcore_axis_name="s")

    @functools.partial(
        pl.kernel,
        out_type=jax.ShapeDtypeStruct((_M, _D), jnp.float32),
        mesh=mesh,
        scratch_types=[
            pltpu.VMEM((_CH,), jnp.int32),
            pltpu.VMEM((_CH, _D), jnp.float32),
            pltpu.SemaphoreType.DMA,
        ],
    )
    def gather_sc(idx_hbm, table_hbm, out_hbm, idx_v, rows_v, sem):
        wid = lax.axis_index("s") * _NC + lax.axis_index("c")
        base = wid * _RPW
        for j in range(_NCH):
            off = base + j * _CH
            pltpu.sync_copy(idx_hbm.at[pl.ds(off, _CH)], idx_v)
            pltpu.async_copy(table_hbm.at[idx_v], rows_v, sem).wait()
            pltpu.sync_copy(rows_v, out_hbm.at[pl.ds(off, _CH)])

    return gather_sc


_BR = 2048
_GR = _M // _BR


def _finish_body(x_ref, p_ref, pst_ref, res_ref, ss_ref):
    i = pl.program_id(0)
    x = x_ref[...]
    p = p_ref[...]
    t = p - x
    pst = x + t
    pst_ref[...] = pst
    res_ref[...] = x - pst

    @pl.when(i == 0)
    def _():
        ss_ref[...] = jnp.zeros_like(ss_ref)

    ss_ref[...] += jnp.sum(t * t).reshape(1, 1)

    @pl.when(i == _GR - 1)
    def _():
        ss_ref[...] = ss_ref[...] * ((1.0 + _COMMIT) / (_M * _D))


_finish = pl.pallas_call(
    _finish_body,
    grid=(_GR,),
    in_specs=[
        pl.BlockSpec((_BR, _D), lambda i: (i, 0)),
        pl.BlockSpec((_BR, _D), lambda i: (i, 0)),
    ],
    out_specs=[
        pl.BlockSpec((_BR, _D), lambda i: (i, 0)),
        pl.BlockSpec((_BR, _D), lambda i: (i, 0)),
        pl.BlockSpec((1, 1), lambda i: (0, 0)),
    ],
    out_shape=[
        jax.ShapeDtypeStruct((_M, _D), jnp.float32),
        jax.ShapeDtypeStruct((_M, _D), jnp.float32),
        jax.ShapeDtypeStruct((1, 1), jnp.float32),
    ],
)


def kernel(x, prototypes):
    x_shape = x.shape
    xf = x.reshape(_M, _D)
    idx = _dist_argmin(xf, prototypes).reshape(_M)
    proto = _build_gather_sc()(idx, prototypes)
    pst, res, loss = _finish(xf, proto)
    return (pst.reshape(x_shape), res.reshape(x_shape), loss.reshape(()))


# speedup vs baseline: 14.4881x; 1.0598x over previous
import functools

import jax
import jax.numpy as jnp
from jax import lax
from jax.experimental import pallas as pl
from jax.experimental.pallas import tpu as pltpu
from jax.experimental.pallas import tpu_sc as plsc

_D = 256
_NP = 8192
_M = 64 * 576
_COMMIT = 0.25

_NCHUNK = 4
_MC = _M // _NCHUNK

_BM = 256
_CG = _MC // _BM


def _dist_argmin_body(x_ref, p_ref, idx_ref, pn_ref):
    i = pl.program_id(0)

    @pl.when(i == 0)
    def _():
        p = p_ref[...]
        pn_ref[...] = jnp.sum(p * p, axis=1)[None, :]

    x = x_ref[...]
    xn = jnp.sum(x * x, axis=1, keepdims=True)
    mm = lax.dot_general(x * -2.0, p_ref[...], (((1,), (1,)), ((), ())),
                         preferred_element_type=jnp.float32)
    dist = (xn + pn_ref[...]) + mm
    idx = jnp.argmin(dist, axis=1).astype(jnp.int32)
    idx_ref[...] = idx[None, None, :]


def _build_dist_argmin(c):
    return pl.pallas_call(
        _dist_argmin_body,
        grid=(_CG,),
        in_specs=[
            pl.BlockSpec((_BM, _D), lambda i, c=c: (c * _CG + i, 0)),
            pl.BlockSpec((_NP, _D), lambda i: (0, 0)),
        ],
        out_specs=pl.BlockSpec((1, 1, _BM), lambda i: (i, 0, 0)),
        out_shape=jax.ShapeDtypeStruct((_CG, 1, _BM), jnp.int32),
        scratch_shapes=[pltpu.VMEM((1, _NP), jnp.float32)],
    )


_dist_calls = [_build_dist_argmin(c) for c in range(_NCHUNK)]

_NC, _NS = 2, 16
_NW = _NC * _NS
_RPW = _MC // _NW
_CH = 96
_NCH = _RPW // _CH


@functools.cache
def _build_gather_sc():
    mesh = plsc.VectorSubcoreMesh(core_axis_name="c", subcore_axis_name="s")

    @functools.partial(
        pl.kernel,
        out_type=jax.ShapeDtypeStruct((_MC, _D), jnp.float32),
        mesh=mesh,
        scratch_types=[
            pltpu.VMEM((_CH,), jnp.int32),
            pltpu.VMEM((_CH, _D), jnp.float32),
            pltpu.SemaphoreType.DMA,
        ],
    )
    def gather_sc(idx_hbm, table_hbm, out_hbm, idx_v, rows_v, sem):
        wid = lax.axis_index("s") * _NC + lax.axis_index("c")
        base = wid * _RPW
        for j in range(_NCH):
            off = base + j * _CH
            pltpu.sync_copy(idx_hbm.at[pl.ds(off, _CH)], idx_v)
            pltpu.async_copy(table_hbm.at[idx_v], rows_v, sem).wait()
            pltpu.sync_copy(rows_v, out_hbm.at[pl.ds(off, _CH)])

    return gather_sc


_BR = 2304
_CGR = _MC // _BR


def _finish_body(c, is_last, x_ref, p_ref, *refs):
    first = len(refs) == 3
    ss_in = None if first else refs[2]
    pst_ref, res_ref, ss_ref = refs[-3:]
    i = pl.program_id(0)
    x = x_ref[...]
    p = p_ref[...]
    t = p - x
    pst = x + t
    pst_ref[...] = pst
    res_ref[...] = x - pst
    part = jnp.sum(t * t).reshape(1, 1)

    @pl.when(i == 0)
    def _():
        ss_ref[...] = part if first else ss_in[...] + part

    @pl.when(i > 0)
    def _():
        ss_ref[...] += part

    if is_last:
        @pl.when(i == _CGR - 1)
        def _():
            ss_ref[...] *= (1.0 + _COMMIT) / (_M * _D)


def _build_finish(c):
    first = c == 0
    is_last = c == _NCHUNK - 1
    win = lambda i, c=c: (c * _CGR + i, 0)
    in_specs = [
        pl.BlockSpec((_BR, _D), win),
        pl.BlockSpec((_BR, _D), lambda i: (i, 0)),
    ]
    if not first:
        in_specs += [
            pl.BlockSpec((8, 128), lambda i: (0, 0)),
            pl.BlockSpec((8, 128), lambda i: (0, 0)),
            pl.BlockSpec((1, 1), lambda i: (0, 0)),
        ]
    return pl.pallas_call(
        functools.partial(_finish_body, c, is_last),
        grid=(_CGR,),
        in_specs=in_specs,
        out_specs=[
            pl.BlockSpec((_BR, _D), win),
            pl.BlockSpec((_BR, _D), win),
            pl.BlockSpec((1, 1), lambda i: (0, 0)),
        ],
        out_shape=[
            jax.ShapeDtypeStruct((_M, _D), jnp.float32),
            jax.ShapeDtypeStruct((_M, _D), jnp.float32),
            jax.ShapeDtypeStruct((1, 1), jnp.float32),
        ],
        input_output_aliases={} if first else {2: 0, 3: 1, 4: 2},
    )


_finish_calls = [_build_finish(c) for c in range(_NCHUNK)]


def kernel(x, prototypes):
    x_shape = x.shape
    xf = x.reshape(_M, _D)
    gather = _build_gather_sc()
    proto_chunks = []
    for c in range(_NCHUNK):
        idx_c = _dist_calls[c](xf, prototypes).reshape(_MC)
        proto_chunks.append(gather(idx_c, prototypes))
    pst, res, ss = _finish_calls[0](xf, proto_chunks[0])
    for c in range(1, _NCHUNK):
        pst, res, ss = _finish_calls[c](xf, proto_chunks[c], pst, res, ss)
    return (pst.reshape(x_shape), res.reshape(x_shape), ss.reshape(()))


# speedup vs baseline: 15.7132x; 1.0846x over previous
import functools

import jax
import jax.numpy as jnp
from jax import lax
from jax.experimental import pallas as pl
from jax.experimental.pallas import tpu as pltpu
from jax.experimental.pallas import tpu_sc as plsc

_D = 256
_NP = 8192
_M = 64 * 576
_COMMIT = 0.25

_NCHUNK = 4
_MC = _M // _NCHUNK

_BM = 512
_CG = _MC // _BM


def _pnorm_body(p_ref, pn_ref):
    p = p_ref[...]
    pn_ref[...] = jnp.sum(p * p, axis=1)[None, :]


_pnorm = pl.pallas_call(
    _pnorm_body,
    in_specs=[pl.BlockSpec((_NP, _D), lambda: (0, 0))],
    out_specs=pl.BlockSpec((1, _NP), lambda: (0, 0)),
    out_shape=jax.ShapeDtypeStruct((1, _NP), jnp.float32),
)


def _dist_argmin_body(x_ref, p_ref, pn_ref, idx_ref):
    x = x_ref[...]
    xn = jnp.sum(x * x, axis=1, keepdims=True)
    mm = lax.dot_general(x * -2.0, p_ref[...], (((1,), (1,)), ((), ())),
                         preferred_element_type=jnp.float32)
    dist = (xn + pn_ref[...]) + mm
    idx = jnp.argmin(dist, axis=1).astype(jnp.int32)
    idx_ref[...] = idx[None, None, :]


def _build_dist_argmin(c):
    return pl.pallas_call(
        _dist_argmin_body,
        grid=(_CG,),
        in_specs=[
            pl.BlockSpec((_BM, _D), lambda i, c=c: (c * _CG + i, 0)),
            pl.BlockSpec((_NP, _D), lambda i: (0, 0)),
            pl.BlockSpec((1, _NP), lambda i: (0, 0)),
        ],
        out_specs=pl.BlockSpec((1, 1, _BM), lambda i: (i, 0, 0)),
        out_shape=jax.ShapeDtypeStruct((_CG, 1, _BM), jnp.int32),
    )


_dist_calls = [_build_dist_argmin(c) for c in range(_NCHUNK)]

_NC, _NS = 2, 16
_NW = _NC * _NS
_RPW = _MC // _NW
_CH = 96
_NCH = _RPW // _CH


@functools.cache
def _build_gather_sc():
    mesh = plsc.VectorSubcoreMesh(core_axis_name="c", subcore_axis_name="s")

    @functools.partial(
        pl.kernel,
        out_type=jax.ShapeDtypeStruct((_MC, _D), jnp.float32),
        mesh=mesh,
        scratch_types=[
            pltpu.VMEM((_CH,), jnp.int32),
            pltpu.VMEM((_CH, _D), jnp.float32),
            pltpu.SemaphoreType.DMA,
        ],
    )
    def gather_sc(idx_hbm, table_hbm, out_hbm, idx_v, rows_v, sem):
        wid = lax.axis_index("s") * _NC + lax.axis_index("c")
        base = wid * _RPW
        for j in range(_NCH):
            off = base + j * _CH
            pltpu.sync_copy(idx_hbm.at[pl.ds(off, _CH)], idx_v)
            pltpu.async_copy(table_hbm.at[idx_v], rows_v, sem).wait()
            pltpu.sync_copy(rows_v, out_hbm.at[pl.ds(off, _CH)])

    return gather_sc


_BR = 2304
_CGR = _MC // _BR


def _finish_body(c, is_last, x_ref, p_ref, *refs):
    first = len(refs) == 3
    ss_in = None if first else refs[2]
    pst_ref, res_ref, ss_ref = refs[-3:]
    i = pl.program_id(0)
    x = x_ref[...]
    p = p_ref[...]
    t = p - x
    pst = x + t
    pst_ref[...] = pst
    res_ref[...] = x - pst
    part = jnp.sum(t * t).reshape(1, 1)

    @pl.when(i == 0)
    def _():
        ss_ref[...] = part if first else ss_in[...] + part

    @pl.when(i > 0)
    def _():
        ss_ref[...] += part

    if is_last:
        @pl.when(i == _CGR - 1)
        def _():
            ss_ref[...] *= (1.0 + _COMMIT) / (_M * _D)


def _build_finish(c):
    first = c == 0
    is_last = c == _NCHUNK - 1
    win = lambda i, c=c: (c * _CGR + i, 0)
    in_specs = [
        pl.BlockSpec((_BR, _D), win),
        pl.BlockSpec((_BR, _D), lambda i: (i, 0)),
    ]
    if not first:
        in_specs += [
            pl.BlockSpec((8, 128), lambda i: (0, 0)),
            pl.BlockSpec((8, 128), lambda i: (0, 0)),
            pl.BlockSpec((1, 1), lambda i: (0, 0)),
        ]
    return pl.pallas_call(
        functools.partial(_finish_body, c, is_last),
        grid=(_CGR,),
        in_specs=in_specs,
        out_specs=[
            pl.BlockSpec((_BR, _D), win),
            pl.BlockSpec((_BR, _D), win),
            pl.BlockSpec((1, 1), lambda i: (0, 0)),
        ],
        out_shape=[
            jax.ShapeDtypeStruct((_M, _D), jnp.float32),
            jax.ShapeDtypeStruct((_M, _D), jnp.float32),
            jax.ShapeDtypeStruct((1, 1), jnp.float32),
        ],
        input_output_aliases={} if first else {2: 0, 3: 1, 4: 2},
    )


_finish_calls = [_build_finish(c) for c in range(_NCHUNK)]


def kernel(x, prototypes):
    x_shape = x.shape
    xf = x.reshape(_M, _D)
    gather = _build_gather_sc()
    pn = _pnorm(prototypes)
    proto_chunks = []
    for c in range(_NCHUNK):
        idx_c = _dist_calls[c](xf, prototypes, pn).reshape(_MC)
        proto_chunks.append(gather(idx_c, prototypes))
    pst, res, ss = _finish_calls[0](xf, proto_chunks[0])
    for c in range(1, _NCHUNK):
        pst, res, ss = _finish_calls[c](xf, proto_chunks[c], pst, res, ss)
    return (pst.reshape(x_shape), res.reshape(x_shape), ss.reshape(()))
